# baseline (speedup 1.0000x reference)
import jax
import jax.numpy as jnp
from jax.experimental import pallas as pl
from jax.experimental.pallas import tpu as pltpu

B = 16
H = 32
W = 32
C = 768
HEADS = 8
HD = C // HEADS
HDP = 128
CP = HEADS * HDP
WIN = 8
NH = H // WIN
NW = W // WIN
NR = NH * NW
W2 = WIN * WIN
TOPK = 4
KS = 5
SCALE = HD ** -0.5


def _route_body(x_ref, w_ref, b_ref, r_ref):
    x16 = x_ref[0].astype(jnp.bfloat16)
    tok = jax.lax.broadcasted_iota(jnp.int32, (NR, H * W), 1)
    rid = (tok // (W * WIN)) * NW + ((tok % W) // WIN)
    row = jax.lax.broadcasted_iota(jnp.int32, (NR, H * W), 0)
    e16 = (rid == row).astype(jnp.bfloat16)
    xm = jnp.dot(e16, x16,
                 preferred_element_type=jnp.float32) * (1.0 / W2)
    qkm = jnp.dot(xm, w_ref[...], preferred_element_type=jnp.float32,
                  precision=jax.lax.Precision.HIGHEST)
    qkm = qkm + b_ref[...]
    qm = qkm[:, :C].astype(jnp.bfloat16)
    km = qkm[:, C:].astype(jnp.bfloat16)
    a = jax.lax.dot_general(qm, km, (((1,), (1,)), ((), ())),
                            preferred_element_type=jnp.float32)
    col = jax.lax.broadcasted_iota(jnp.int32, (NR, NR), 1)
    rows = []
    work = a
    for _ in range(TOPK):
        mx = jnp.max(work, axis=1, keepdims=True)
        idx_t = jnp.min(jnp.where(work == mx, col, NR), axis=1)
        work = jnp.where(col == idx_t[:, None], -1e30, work)
        rows.append(idx_t[None, :])
    rows.append(jnp.zeros((8 - TOPK, NR), jnp.int32))
    r_ref[0] = jnp.concatenate(rows, axis=0)


def _route_call(x3, Wqk, bqk):
    return pl.pallas_call(
        _route_body,
        grid=(B,),
        in_specs=[
            pl.BlockSpec((1, H * W, C), lambda b: (b, 0, 0)),
            pl.BlockSpec((C, 2 * C), lambda b: (0, 0)),
            pl.BlockSpec((1, 2 * C), lambda b: (0, 0)),
        ],
        out_specs=pl.BlockSpec((1, 8, NR), lambda b: (b, 0, 0)),
        out_shape=jax.ShapeDtypeStruct((B, 8, NR), jnp.int32),
    )(x3, Wqk, bqk)


_HP = H + KS
_WP = W + KS - 1
_FP = _HP * _WP
_FV = H * _WP
_CH = 8


def _mega_body(r_ref, x_ref, w_ref, b_ref, t_ref, lb_ref, wo_ref, bo_ref,
               o_ref, k_s, v_s, pad_ref):
    b = pl.program_id(0)
    x16 = x_ref[0].astype(jnp.bfloat16)
    shp5 = (NH, WIN, NW, WIN, CP)

    qf = jnp.dot(x16, w_ref[:, :CP],
                 preferred_element_type=jnp.float32) + b_ref[:, :CP]
    q16 = qf.astype(jnp.bfloat16).reshape(shp5)
    kf = jnp.dot(x16, w_ref[:, CP:2 * CP],
                 preferred_element_type=jnp.float32) + b_ref[:, CP:2 * CP]
    k_s[...] = kf.astype(jnp.bfloat16).reshape(shp5)
    vf = jnp.dot(x16, w_ref[:, 2 * CP:],
                 preferred_element_type=jnp.float32) + b_ref[:, 2 * CP:]
    v16 = vf.astype(jnp.bfloat16)
    v_s[...] = v16.reshape(shp5)

    pad_ref[...] = jnp.zeros((_FP, HEADS, HDP), jnp.bfloat16)
    v4 = v16.reshape(H * W, HEADS, HDP)
    for y in range(H):
        base = (y + 2) * _WP + 2
        pad_ref[base:base + W] = v4[y * W:(y + 1) * W]
    xs = pad_ref[...]
    acc = jnp.zeros((_FV, HEADS, HDP), jnp.float32) + lb_ref[0]
    for dy in range(KS):
        for dx in range(KS):
            s = dy * _WP + dx
            acc += xs[s:s + _FV].astype(jnp.float32) * t_ref[dy * KS + dx]
    lepe = (acc.reshape(H, _WP, HEADS, HDP)[:, :W]
            .reshape(H * W, CP).astype(jnp.bfloat16))

    att = []
    for c0 in range(0, NR, _CH):
        qs, kgs, vgs = [], [], []
        for r in range(c0, c0 + _CH):
            ii, j = r // NW, r % NW
            qs.append(q16[ii, :, j].reshape(W2, CP))
            kts, vts = [], []
            for t in range(TOPK):
                ri = r_ref[b, t, r]
                di, dj = ri // NW, ri % NW
                kts.append(k_s[di, :, dj].reshape(W2, CP))
                vts.append(v_s[di, :, dj].reshape(W2, CP))
            kgs.append(jnp.concatenate(kts, axis=0))
            vgs.append(jnp.concatenate(vts, axis=0))
        ss = []
        for i in range(_CH):
            for h in range(HEADS):
                sl = slice(h * HDP, (h + 1) * HDP)
                ss.append(jax.lax.dot_general(
                    qs[i][:, sl], kgs[i][:, sl], (((1,), (1,)), ((), ())),
                    preferred_element_type=jnp.float32))
        ps = []
        for s in ss:
            s = s * SCALE
            s = s - jnp.max(s, axis=1, keepdims=True)
            p = jnp.exp(s)
            p = p / jnp.sum(p, axis=1, keepdims=True)
            ps.append(p.astype(jnp.bfloat16))
        for i in range(_CH):
            outs = []
            for h in range(HEADS):
                sl = slice(h * HDP, (h + 1) * HDP)
                outs.append(jnp.dot(ps[i * HEADS + h], vgs[i][:, sl],
                                    preferred_element_type=jnp.float32))
            att.append(jnp.concatenate(outs, axis=1).astype(jnp.bfloat16))

    a_img = jnp.concatenate(
        [att[(y // WIN) * NW + j][(y % WIN) * WIN:(y % WIN + 1) * WIN]
         for y in range(H) for j in range(NW)], axis=0)
    ssum = a_img + lepe
    o_ref[0] = jnp.dot(ssum, wo_ref[...],
                       preferred_element_type=jnp.float32) + bo_ref[...]


def _mega_call(r_idx, x3, Wp16, bp, taps, lb, Wop16, bo):
    grid_spec = pltpu.PrefetchScalarGridSpec(
        num_scalar_prefetch=1,
        grid=(B,),
        in_specs=[
            pl.BlockSpec((1, H * W, C), lambda b, rr: (b, 0, 0)),
            pl.BlockSpec((C, 3 * CP), lambda b, rr: (0, 0)),
            pl.BlockSpec((1, 3 * CP), lambda b, rr: (0, 0)),
            pl.BlockSpec((KS * KS, HEADS, HDP), lambda b, rr: (0, 0, 0)),
            pl.BlockSpec((1, HEADS, HDP), lambda b, rr: (0, 0, 0)),
            pl.BlockSpec((CP, C), lambda b, rr: (0, 0)),
            pl.BlockSpec((1, C), lambda b, rr: (0, 0)),
        ],
        out_specs=pl.BlockSpec((1, H * W, C), lambda b, rr: (b, 0, 0)),
        scratch_shapes=[
            pltpu.VMEM((NH, WIN, NW, WIN, CP), jnp.bfloat16),
            pltpu.VMEM((NH, WIN, NW, WIN, CP), jnp.bfloat16),
            pltpu.VMEM((_FP, HEADS, HDP), jnp.bfloat16),
        ],
    )
    return pl.pallas_call(
        _mega_body,
        grid_spec=grid_spec,
        out_shape=jax.ShapeDtypeStruct((B, H * W, C), jnp.float32),
    )(r_idx, x3, Wp16, bp, taps, lb, Wop16, bo)


def _pad_heads(t):
    lead = t.shape[:-1]
    n = t.shape[-1] // HD
    t = t.reshape(lead + (n, HD))
    t = jnp.pad(t, [(0, 0)] * len(lead) + [(0, 0), (0, HDP - HD)])
    return t.reshape(lead + (n * HDP,))


@jax.jit
def _run(x, Wqkv, bqkv, Wo, bo, lepe_w, lepe_b):
    Wp16 = _pad_heads(Wqkv).astype(jnp.bfloat16)
    bp = _pad_heads(bqkv).reshape(1, 3 * CP)
    Wop16 = (jnp.pad(Wo.reshape(HEADS, HD, C),
                     ((0, 0), (0, HDP - HD), (0, 0)))
             .reshape(CP, C).astype(jnp.bfloat16))
    taps = _pad_heads(lepe_w.reshape(C, KS * KS).T).reshape(KS * KS, HEADS, HDP)
    lb = _pad_heads(lepe_b).reshape(1, HEADS, HDP)
    Wqk = Wqkv[:, :2 * C].astype(jnp.bfloat16).astype(jnp.float32)
    bqk = bqkv[:2 * C].reshape(1, 2 * C)

    r_idx = _route_call(x, Wqk, bqk)
    out = _mega_call(r_idx, x, Wp16, bp, taps, lb, Wop16,
                     bo.reshape(1, C))
    return out


def kernel(x, x_size, Wqkv, bqkv, Wo, bo, lepe_w, lepe_b):
    return _run(x, Wqkv, bqkv, Wo, bo, lepe_w, lepe_b)


# speedup vs baseline: 1.0916x; 1.0916x over previous
import jax
import jax.numpy as jnp
from jax.experimental import pallas as pl
from jax.experimental.pallas import tpu as pltpu

B = 16
H = 32
W = 32
C = 768
HEADS = 8
HD = C // HEADS
HDP = 128
CP = HEADS * HDP
WIN = 8
NH = H // WIN
NW = W // WIN
NR = NH * NW
W2 = WIN * WIN
TOPK = 4
KS = 5
SCALE = HD ** -0.5


def _qkv_body(x_ref, w_ref, b_ref, q_ref, k_ref, v_ref, r_ref):
    x16 = x_ref[0].astype(jnp.bfloat16)
    parts = []
    for i, o_ref in enumerate((q_ref, k_ref, v_ref)):
        sl = slice(i * CP, (i + 1) * CP)
        p = jnp.dot(x16, w_ref[:, sl], preferred_element_type=jnp.float32)
        p = p + b_ref[:, sl]
        o_ref[0] = p.astype(jnp.bfloat16)
        parts.append(p)
    qf, kf = parts[0], parts[1]

    def rmean(t):
        t6 = t.reshape(NH, WIN, NW, WIN, CP)
        return t6.sum(axis=3).sum(axis=1).reshape(NR, CP) * (1.0 / W2)

    qm = rmean(qf).astype(jnp.bfloat16)
    km = rmean(kf).astype(jnp.bfloat16)
    a = jax.lax.dot_general(qm, km, (((1,), (1,)), ((), ())),
                            preferred_element_type=jnp.float32)
    col = jax.lax.broadcasted_iota(jnp.int32, (NR, NR), 1)
    rows = []
    work = a
    for _ in range(TOPK):
        mx = jnp.max(work, axis=1, keepdims=True)
        idx_t = jnp.min(jnp.where(work == mx, col, NR), axis=1)
        work = jnp.where(col == idx_t[:, None], -1e30, work)
        rows.append(idx_t[None, :])
    rows.append(jnp.zeros((8 - TOPK, NR), jnp.int32))
    r_ref[0] = jnp.concatenate(rows, axis=0)


def _qkv_call(x3, Wp16, bp):
    out = jax.ShapeDtypeStruct((B, H * W, CP), jnp.bfloat16)
    return pl.pallas_call(
        _qkv_body,
        grid=(B,),
        in_specs=[
            pl.BlockSpec((1, H * W, C), lambda b: (b, 0, 0)),
            pl.BlockSpec((C, 3 * CP), lambda b: (0, 0)),
            pl.BlockSpec((1, 3 * CP), lambda b: (0, 0)),
        ],
        out_specs=[
            pl.BlockSpec((1, H * W, CP), lambda b: (b, 0, 0)),
            pl.BlockSpec((1, H * W, CP), lambda b: (b, 0, 0)),
            pl.BlockSpec((1, H * W, CP), lambda b: (b, 0, 0)),
            pl.BlockSpec((1, 8, NR), lambda b: (b, 0, 0)),
        ],
        out_shape=[out, out, out,
                   jax.ShapeDtypeStruct((B, 8, NR), jnp.int32)],
    )(x3, Wp16, bp)


_HP = H + KS
_WP = W + KS - 1
_FP = _HP * _WP
_FV = H * _WP
_CH = 8


def _fuse_body(r_ref, q_ref, k_ref, v_ref, t_ref, lb_ref, w_ref, bo_ref,
               o_ref, pad_ref):
    b = pl.program_id(0)
    att = []
    for c0 in range(0, NR, _CH):
        qs, kgs, vgs = [], [], []
        for r in range(c0, c0 + _CH):
            ii, j = r // NW, r % NW
            qs.append(q_ref[0, ii, :, j].reshape(W2, CP))
            kts, vts = [], []
            for t in range(TOPK):
                ri = r_ref[b, t, r]
                di, dj = ri // NW, ri % NW
                kts.append(k_ref[0, di, :, dj].reshape(W2, CP))
                vts.append(v_ref[0, di, :, dj].reshape(W2, CP))
            kgs.append(jnp.concatenate(kts, axis=0))
            vgs.append(jnp.concatenate(vts, axis=0))
        ss = []
        for i in range(_CH):
            for h in range(HEADS):
                sl = slice(h * HDP, (h + 1) * HDP)
                ss.append(jax.lax.dot_general(
                    qs[i][:, sl], kgs[i][:, sl], (((1,), (1,)), ((), ())),
                    preferred_element_type=jnp.float32))
        ps = []
        for s in ss:
            s = s * SCALE
            s = s - jnp.max(s, axis=1, keepdims=True)
            p = jnp.exp(s)
            p = p / jnp.sum(p, axis=1, keepdims=True)
            ps.append(p.astype(jnp.bfloat16))
        for i in range(_CH):
            outs = []
            for h in range(HEADS):
                sl = slice(h * HDP, (h + 1) * HDP)
                outs.append(jnp.dot(ps[i * HEADS + h], vgs[i][:, sl],
                                    preferred_element_type=jnp.float32))
            att.append(jnp.concatenate(outs, axis=1).astype(jnp.bfloat16))
    pad_ref[...] = jnp.zeros((_FP, HEADS, HDP), jnp.bfloat16)
    v4 = v_ref[0].reshape(H * W, HEADS, HDP)
    for y in range(H):
        base = (y + 2) * _WP + 2
        pad_ref[base:base + W] = v4[y * W:(y + 1) * W]
    xs = pad_ref[...]
    acc = jnp.zeros((_FV, HEADS, HDP), jnp.float32) + lb_ref[0]
    for dy in range(KS):
        for dx in range(KS):
            s = dy * _WP + dx
            acc += xs[s:s + _FV].astype(jnp.float32) * t_ref[dy * KS + dx]
    lepe = (acc.reshape(H, _WP, HEADS, HDP)[:, :W]
            .reshape(H * W, CP).astype(jnp.bfloat16))
    a_img = jnp.concatenate(
        [att[(y // WIN) * NW + j][(y % WIN) * WIN:(y % WIN + 1) * WIN]
         for y in range(H) for j in range(NW)], axis=0)
    ssum = a_img + lepe
    o_ref[0] = jnp.dot(ssum, w_ref[...],
                       preferred_element_type=jnp.float32) + bo_ref[...]


def _fuse_call(r_idx, q6, k6, v6, taps, lb, Wop16, bo):
    kv_blk = (1, NH, WIN, NW, WIN, CP)

    def kv_map(b, rr):
        return (b, 0, 0, 0, 0, 0)

    grid_spec = pltpu.PrefetchScalarGridSpec(
        num_scalar_prefetch=1,
        grid=(B,),
        in_specs=[
            pl.BlockSpec(kv_blk, kv_map),
            pl.BlockSpec(kv_blk, kv_map),
            pl.BlockSpec(kv_blk, kv_map),
            pl.BlockSpec((KS * KS, HEADS, HDP), lambda b, rr: (0, 0, 0)),
            pl.BlockSpec((1, HEADS, HDP), lambda b, rr: (0, 0, 0)),
            pl.BlockSpec((CP, C), lambda b, rr: (0, 0)),
            pl.BlockSpec((1, C), lambda b, rr: (0, 0)),
        ],
        out_specs=pl.BlockSpec((1, H * W, C), lambda b, rr: (b, 0, 0)),
        scratch_shapes=[pltpu.VMEM((_FP, HEADS, HDP), jnp.bfloat16)],
    )
    return pl.pallas_call(
        _fuse_body,
        grid_spec=grid_spec,
        out_shape=jax.ShapeDtypeStruct((B, H * W, C), jnp.float32),
    )(r_idx, q6, k6, v6, taps, lb, Wop16, bo)


def _pad_heads(t):
    lead = t.shape[:-1]
    n = t.shape[-1] // HD
    t = t.reshape(lead + (n, HD))
    t = jnp.pad(t, [(0, 0)] * len(lead) + [(0, 0), (0, HDP - HD)])
    return t.reshape(lead + (n * HDP,))


@jax.jit
def _run(x, Wqkv, bqkv, Wo, bo, lepe_w, lepe_b):
    Wp16 = _pad_heads(Wqkv).astype(jnp.bfloat16)
    bp = _pad_heads(bqkv).reshape(1, 3 * CP)
    Wop16 = (jnp.pad(Wo.reshape(HEADS, HD, C),
                     ((0, 0), (0, HDP - HD), (0, 0)))
             .reshape(CP, C).astype(jnp.bfloat16))
    taps = _pad_heads(lepe_w.reshape(C, KS * KS).T).reshape(KS * KS, HEADS, HDP)
    lb = _pad_heads(lepe_b).reshape(1, HEADS, HDP)

    q, k, v, r_idx = _qkv_call(x, Wp16, bp)
    shp6 = (B, NH, WIN, NW, WIN, CP)
    out = _fuse_call(r_idx, q.reshape(shp6), k.reshape(shp6),
                     v.reshape(shp6), taps, lb, Wop16, bo.reshape(1, C))
    return out


def kernel(x, x_size, Wqkv, bqkv, Wo, bo, lepe_w, lepe_b):
    return _run(x, Wqkv, bqkv, Wo, bo, lepe_w, lepe_b)
